# baseline (speedup 1.0000x reference)
import functools

import jax
import jax.numpy as jnp
from jax import lax
from jax.experimental import pallas as pl
from jax.experimental.pallas import tpu as pltpu
from jax.experimental.pallas import tpu_sc as plsc

VOCAB = 1000
EMBED = 32
BATCH = 4096
HIST = 200

N = BATCH * HIST
NWORKERS = 32
IPW = N // NWORKERS
CHI = 256
NCHUNK = IPW // CHI
NS = 4
CHF = CHI * EMBED

_mesh = plsc.VectorSubcoreMesh(core_axis_name="c", subcore_axis_name="s")


@functools.partial(
    pl.kernel,
    mesh=_mesh,
    out_type=jax.ShapeDtypeStruct((N * EMBED,), jnp.float32),
    scratch_types=[
        pltpu.VMEM((IPW,), jnp.int32),
        pltpu.VMEM((VOCAB * EMBED,), jnp.float32),
        pltpu.VMEM((NS, CHF), jnp.float32),
        pltpu.SemaphoreType.DMA((NS,)),
    ],
    compiler_params=pltpu.CompilerParams(
        use_tc_tiling_on_sc=False, needs_layout_passes=False
    ),
)
def _gather_kernel(idx_hbm, table_hbm, out_hbm, idx_v, table_v, ring, out_sems):
    wid = lax.axis_index("s") * 2 + lax.axis_index("c")
    ibase = wid * IPW
    obase = ibase * EMBED
    pltpu.sync_copy(table_hbm, table_v)
    pltpu.sync_copy(idx_hbm.at[pl.ds(ibase, IPW)], idx_v)

    lane = lax.iota(jnp.int32, 16)
    pos32 = lane * EMBED

    def chunk_body(j, carry):
        s = j % NS

        @pl.when(j >= NS)
        def _slot_free():
            pltpu.make_async_copy(
                ring.at[s], out_hbm.at[pl.ds(0, CHF)], out_sems.at[s]
            ).wait()

        for g in range(CHI // 16):
            rows = idx_v[pl.ds(j * CHI + g * 16, 16)]
            gbase = rows * EMBED
            sbase = pos32 + (g * 16 * EMBED)
            for c in range(EMBED):
                vals = plsc.load_gather(table_v, [gbase + c])
                plsc.store_scatter(ring.at[s], [sbase + c], vals)

        pltpu.async_copy(
            ring.at[s],
            out_hbm.at[pl.ds(obase + j * CHF, CHF)],
            out_sems.at[s],
        )
        return carry

    lax.fori_loop(0, NCHUNK, chunk_body, 0)
    for s in range(NS):
        pltpu.make_async_copy(
            ring.at[s], out_hbm.at[pl.ds(0, CHF)], out_sems.at[s]
        ).wait()


def kernel(word_idx_list, W):
    idx = word_idx_list.astype(jnp.int32).reshape(N)
    out = _gather_kernel(idx, W.reshape(VOCAB * EMBED))
    return out.reshape(BATCH, HIST, EMBED)


# speedup vs baseline: 1.5548x; 1.5548x over previous
import functools

import jax
import jax.numpy as jnp
from jax import lax
from jax.experimental import pallas as pl
from jax.experimental.pallas import tpu as pltpu
from jax.experimental.pallas import tpu_sc as plsc

VOCAB = 1000
EMBED = 32
BATCH = 4096
HIST = 200

N = BATCH * HIST
NWORKERS = 32
IPW = N // NWORKERS
CHI = 256
NCHUNK = IPW // CHI
NS = 4
CHF = CHI * EMBED

_mesh = plsc.VectorSubcoreMesh(core_axis_name="c", subcore_axis_name="s")


@functools.partial(
    pl.kernel,
    mesh=_mesh,
    out_type=jax.ShapeDtypeStruct((N, EMBED), jnp.float32),
    scratch_types=[
        pltpu.VMEM((IPW,), jnp.int32),
        pltpu.VMEM((VOCAB * EMBED,), jnp.float32),
        pltpu.VMEM((NS, CHI, EMBED), jnp.float32),
        pltpu.SemaphoreType.DMA((NS,)),
    ],
    compiler_params=pltpu.CompilerParams(
        use_tc_tiling_on_sc=False, needs_layout_passes=False
    ),
)
def _gather_kernel(idx_hbm, table_hbm, out_hbm, idx_v, table_v, ring, out_sems):
    wid = lax.axis_index("s") * 2 + lax.axis_index("c")
    ibase = wid * IPW
    pltpu.sync_copy(table_hbm, table_v)
    pltpu.sync_copy(idx_hbm.at[pl.ds(ibase, IPW)], idx_v)

    lane = lax.iota(jnp.int32, 16)

    def chunk_body(j, carry):
        s = j % NS

        @pl.when(j >= NS)
        def _slot_free():
            pltpu.make_async_copy(
                ring.at[s], out_hbm.at[pl.ds(0, CHI)], out_sems.at[s]
            ).wait()

        for p in range(CHI):
            spl = plsc.load_gather(idx_v, [jnp.full((16,), j * CHI + p, jnp.int32)])
            addr0 = spl * EMBED + lane
            v0 = plsc.load_gather(table_v, [addr0])
            v1 = plsc.load_gather(table_v, [addr0 + 16])
            ring[s, p, pl.ds(0, 16)] = v0
            ring[s, p, pl.ds(16, 16)] = v1

        pltpu.async_copy(
            ring.at[s],
            out_hbm.at[pl.ds(ibase + j * CHI, CHI)],
            out_sems.at[s],
        )
        return carry

    lax.fori_loop(0, NCHUNK, chunk_body, 0)
    for s in range(NS):
        pltpu.make_async_copy(
            ring.at[s], out_hbm.at[pl.ds(0, CHI)], out_sems.at[s]
        ).wait()


def kernel(word_idx_list, W):
    idx = word_idx_list.astype(jnp.int32).reshape(N)
    out = _gather_kernel(idx, W.reshape(VOCAB * EMBED))
    return out.reshape(BATCH, HIST, EMBED)
